# baseline (speedup 1.0000x reference)
import functools

import jax
import jax.numpy as jnp
from jax import lax
from jax.experimental import pallas as pl
from jax.experimental.pallas import tpu as pltpu
from jax.experimental.pallas import tpu_sc as plsc

BATCH = 16384
DIM = 4
LANES = 16
IDX_CHUNK = 128
NUM_WORKERS = 32
B_PER_W = BATCH // NUM_WORKERS
N_CHUNKS = B_PER_W // IDX_CHUNK
SUP = 16
ROWS_PER_SUP = SUP // DIM


def _rsqrt16(x):
    i = lax.bitcast_convert_type(x, jnp.int32)
    y = lax.bitcast_convert_type(jnp.int32(0x5F3759DF) - (i >> 1), jnp.float32)
    for _ in range(3):
        y = y * (jnp.float32(1.5) - jnp.float32(0.5) * x * y * y)
    return y


mesh = plsc.VectorSubcoreMesh(core_axis_name="c", subcore_axis_name="s")


@functools.partial(
    pl.kernel,
    mesh=mesh,
    out_type=jax.ShapeDtypeStruct((BATCH,), jnp.float32),
    compiler_params=pltpu.CompilerParams(
        needs_layout_passes=False, use_tc_tiling_on_sc=False
    ),
    scratch_types=[
        pltpu.VMEM((N_CHUNKS, IDX_CHUNK), jnp.int32),
        pltpu.VMEM((N_CHUNKS, IDX_CHUNK), jnp.int32),
        pltpu.VMEM((B_PER_W,), jnp.int32),
        pltpu.VMEM((B_PER_W,), jnp.int32),
        pltpu.VMEM((B_PER_W, SUP), jnp.float32),
        pltpu.VMEM((B_PER_W, SUP), jnp.float32),
        pltpu.VMEM((B_PER_W,), jnp.float32),
        pltpu.SemaphoreType.DMA,
    ],
)
def _sc_kernel(uin_hbm, iin_hbm, utab_hbm, itab_hbm, out_hbm,
               uidx, iidx, usup, isup, urows, irows, outv, sem):
    wid = lax.axis_index("s") * 2 + lax.axis_index("c")
    base = wid * B_PER_W

    pltpu.sync_copy(uin_hbm.at[pl.ds(wid * N_CHUNKS, N_CHUNKS)], uidx)
    pltpu.sync_copy(iin_hbm.at[pl.ds(wid * N_CHUNKS, N_CHUNKS)], iidx)

    n_vec = B_PER_W // LANES
    for c in range(n_vec):
        j, k = c // (IDX_CHUNK // LANES), (c % (IDX_CHUNK // LANES)) * LANES
        usup[pl.ds(c * LANES, LANES)] = uidx[j, pl.ds(k, LANES)] >> 2
        isup[pl.ds(c * LANES, LANES)] = iidx[j, pl.ds(k, LANES)] >> 2

    cu = pltpu.make_async_copy(utab_hbm.at[usup], urows, sem)
    ci = pltpu.make_async_copy(itab_hbm.at[isup], irows, sem)
    cu.start()
    ci.start()
    cu.wait()
    ci.wait()

    lane = lax.iota(jnp.int32, LANES)
    for c in range(n_vec):
        j, k = c // (IDX_CHUNK // LANES), (c % (IDX_CHUNK // LANES)) * LANES
        row = c * LANES + lane
        ucol0 = (uidx[j, pl.ds(k, LANES)] & 3) << 2
        icol0 = (iidx[j, pl.ds(k, LANES)] & 3) << 2
        ud, vd = [], []
        for d in range(DIM):
            ud.append(plsc.load_gather(urows, [row, ucol0 + d]))
            vd.append(plsc.load_gather(irows, [row, icol0 + d]))
        dot = ud[0] * vd[0]
        nu = ud[0] * ud[0]
        nv = vd[0] * vd[0]
        for d in range(1, DIM):
            dot = dot + ud[d] * vd[d]
            nu = nu + ud[d] * ud[d]
            nv = nv + vd[d] * vd[d]
        outv[pl.ds(c * LANES, LANES)] = dot * _rsqrt16(nu) * _rsqrt16(nv)

    pltpu.sync_copy(outv, out_hbm.at[pl.ds(base, B_PER_W)])


def kernel(user_input, item_input, user_table, item_table):
    uin = user_input.reshape(BATCH // IDX_CHUNK, IDX_CHUNK)
    iin = item_input.reshape(BATCH // IDX_CHUNK, IDX_CHUNK)
    uflat = jax.lax.optimization_barrier(user_table.reshape(-1))
    iflat = jax.lax.optimization_barrier(item_table.reshape(-1))
    utab = uflat.reshape(user_table.shape[0] // ROWS_PER_SUP, SUP)
    itab = iflat.reshape(item_table.shape[0] // ROWS_PER_SUP, SUP)
    return _sc_kernel(uin, iin, utab, itab)


# speedup vs baseline: 1.0002x; 1.0002x over previous
import functools

import jax
import jax.numpy as jnp
from jax import lax
from jax.experimental import pallas as pl
from jax.experimental.pallas import tpu as pltpu
from jax.experimental.pallas import tpu_sc as plsc

BATCH = 16384
DIM = 4
LANES = 16
NUM_WORKERS = 32
B_PER_W = BATCH // NUM_WORKERS
N_VEC = B_PER_W // LANES


def _rsqrt16(x):
    i = lax.bitcast_convert_type(x, jnp.int32)
    y = lax.bitcast_convert_type(jnp.int32(0x5F3759DF) - (i >> 1), jnp.float32)
    for _ in range(3):
        y = y * (jnp.float32(1.5) - jnp.float32(0.5) * x * y * y)
    return y


mesh = plsc.VectorSubcoreMesh(core_axis_name="c", subcore_axis_name="s")


@functools.partial(
    pl.kernel,
    mesh=mesh,
    out_type=jax.ShapeDtypeStruct((BATCH,), jnp.float32),
    compiler_params=pltpu.CompilerParams(
        needs_layout_passes=False, use_tc_tiling_on_sc=False
    ),
    scratch_types=[
        pltpu.VMEM((B_PER_W,), jnp.int32),
        pltpu.VMEM((B_PER_W,), jnp.int32),
        pltpu.VMEM((DIM * B_PER_W,), jnp.int32),
        pltpu.VMEM((DIM * B_PER_W,), jnp.int32),
        pltpu.VMEM((DIM * B_PER_W,), jnp.float32),
        pltpu.VMEM((DIM * B_PER_W,), jnp.float32),
        pltpu.VMEM((B_PER_W,), jnp.float32),
        pltpu.SemaphoreType.DMA,
    ],
)
def _sc_kernel(uin_hbm, iin_hbm, utab_hbm, itab_hbm, out_hbm,
               uidx, iidx, uidx4, iidx4, ucomp, icomp, outv, sem):
    wid = lax.axis_index("s") * 2 + lax.axis_index("c")
    base = wid * B_PER_W

    pltpu.sync_copy(uin_hbm.at[pl.ds(base, B_PER_W)], uidx)
    pltpu.sync_copy(iin_hbm.at[pl.ds(base, B_PER_W)], iidx)

    for c in range(N_VEC):
        sl = pl.ds(c * LANES, LANES)
        u4 = uidx[sl] << 2
        i4 = iidx[sl] << 2
        for d in range(DIM):
            uidx4[pl.ds(d * B_PER_W + c * LANES, LANES)] = u4 + d
            iidx4[pl.ds(d * B_PER_W + c * LANES, LANES)] = i4 + d

    copies = []
    for d in range(DIM):
        sl = pl.ds(d * B_PER_W, B_PER_W)
        copies.append(pltpu.make_async_copy(
            utab_hbm.at[uidx4.at[sl]], ucomp.at[sl], sem))
        copies.append(pltpu.make_async_copy(
            itab_hbm.at[iidx4.at[sl]], icomp.at[sl], sem))
    for cp in copies:
        cp.start()
    for cp in copies:
        cp.wait()

    for c in range(N_VEC):
        ud = [ucomp[pl.ds(d * B_PER_W + c * LANES, LANES)] for d in range(DIM)]
        vd = [icomp[pl.ds(d * B_PER_W + c * LANES, LANES)] for d in range(DIM)]
        dot = ud[0] * vd[0]
        nu = ud[0] * ud[0]
        nv = vd[0] * vd[0]
        for d in range(1, DIM):
            dot = dot + ud[d] * vd[d]
            nu = nu + ud[d] * ud[d]
            nv = nv + vd[d] * vd[d]
        outv[pl.ds(c * LANES, LANES)] = dot * _rsqrt16(nu) * _rsqrt16(nv)

    pltpu.sync_copy(outv, out_hbm.at[pl.ds(base, B_PER_W)])


def kernel(user_input, item_input, user_table, item_table):
    return _sc_kernel(user_input, item_input,
                      user_table.reshape(-1), item_table.reshape(-1))


# speedup vs baseline: 1.0032x; 1.0030x over previous
import functools

import jax
import jax.numpy as jnp
from jax import lax
from jax.experimental import pallas as pl
from jax.experimental.pallas import tpu as pltpu
from jax.experimental.pallas import tpu_sc as plsc

BATCH = 16384
DIM = 4
LANES = 16
NUM_WORKERS = 32
B_PER_W = BATCH // NUM_WORKERS
N_VEC = B_PER_W // LANES


def _rsqrt16(x):
    i = lax.bitcast_convert_type(x, jnp.int32)
    y = lax.bitcast_convert_type(jnp.int32(0x5F3759DF) - (i >> 1), jnp.float32)
    for _ in range(3):
        y = y * (jnp.float32(1.5) - jnp.float32(0.5) * x * y * y)
    return y


mesh = plsc.VectorSubcoreMesh(core_axis_name="c", subcore_axis_name="s")


@functools.partial(
    pl.kernel,
    mesh=mesh,
    out_type=jax.ShapeDtypeStruct((BATCH,), jnp.float32),
    compiler_params=pltpu.CompilerParams(
        needs_layout_passes=False, use_tc_tiling_on_sc=False
    ),
    scratch_types=[
        pltpu.VMEM((B_PER_W,), jnp.int32),
        pltpu.VMEM((B_PER_W,), jnp.int32),
        pltpu.VMEM((DIM * B_PER_W,), jnp.int32),
        pltpu.VMEM((DIM * B_PER_W,), jnp.int32),
        pltpu.VMEM((DIM * B_PER_W,), jnp.float32),
        pltpu.VMEM((DIM * B_PER_W,), jnp.float32),
        pltpu.VMEM((B_PER_W,), jnp.float32),
        pltpu.SemaphoreType.DMA,
    ],
)
def _sc_kernel(uin_hbm, iin_hbm, utab_hbm, itab_hbm, out_hbm,
               uidx, iidx, uidx4, iidx4, ucomp, icomp, outv, sem):
    wid = lax.axis_index("s") * 2 + lax.axis_index("c")
    base = wid * B_PER_W

    pltpu.sync_copy(uin_hbm.at[pl.ds(base, B_PER_W)], uidx)
    pltpu.sync_copy(iin_hbm.at[pl.ds(base, B_PER_W)], iidx)

    for c in range(N_VEC):
        sl = pl.ds(c * LANES, LANES)
        u4 = uidx[sl] << 2
        i4 = iidx[sl] << 2
        for d in range(DIM):
            uidx4[pl.ds(d * B_PER_W + c * LANES, LANES)] = u4 + d
            iidx4[pl.ds(d * B_PER_W + c * LANES, LANES)] = i4 + d

    copies = []
    for d in range(DIM):
        sl = pl.ds(d * B_PER_W, B_PER_W)
        copies.append(pltpu.make_async_copy(
            utab_hbm.at[uidx4.at[sl]], ucomp.at[sl], sem))
        copies.append(pltpu.make_async_copy(
            itab_hbm.at[iidx4.at[sl]], icomp.at[sl], sem))
    for cp in copies:
        cp.start()
    for cp in copies:
        cp.wait()

    for c in range(N_VEC):
        ud = [ucomp[pl.ds(d * B_PER_W + c * LANES, LANES)] for d in range(DIM)]
        vd = [icomp[pl.ds(d * B_PER_W + c * LANES, LANES)] for d in range(DIM)]
        dot = ud[0] * vd[0]
        nu = ud[0] * ud[0]
        nv = vd[0] * vd[0]
        for d in range(1, DIM):
            dot = dot + ud[d] * vd[d]
            nu = nu + ud[d] * ud[d]
            nv = nv + vd[d] * vd[d]
        outv[pl.ds(c * LANES, LANES)] = dot * _rsqrt16(nu) * _rsqrt16(nv)

    pltpu.sync_copy(outv, out_hbm.at[pl.ds(base, B_PER_W)])


def kernel(user_input, item_input, user_table, item_table):
    one = (user_input[0] * 0 + 1).astype(jnp.float32)
    return _sc_kernel(user_input, item_input,
                      (user_table * one).reshape(-1),
                      (item_table * one).reshape(-1))


# speedup vs baseline: 17.3885x; 17.3329x over previous
import functools

import jax
import jax.numpy as jnp
from jax import lax
from jax.experimental import pallas as pl
from jax.experimental.pallas import tpu as pltpu
from jax.experimental.pallas import tpu_sc as plsc

BATCH = 16384
DIM = 4
LANES = 16
NUM_WORKERS = 32
B_PER_W = BATCH // NUM_WORKERS
N_VEC = B_PER_W // LANES


def _rsqrt16(x):
    i = lax.bitcast_convert_type(x, jnp.int32)
    y = lax.bitcast_convert_type(jnp.int32(0x5F3759DF) - (i >> 1), jnp.float32)
    for _ in range(3):
        y = y * (jnp.float32(1.5) - jnp.float32(0.5) * x * y * y)
    return y


mesh = plsc.VectorSubcoreMesh(core_axis_name="c", subcore_axis_name="s")


@functools.partial(
    pl.kernel,
    mesh=mesh,
    out_type=jax.ShapeDtypeStruct((BATCH,), jnp.float32),
    compiler_params=pltpu.CompilerParams(
        needs_layout_passes=False, use_tc_tiling_on_sc=False
    ),
    scratch_types=[
        pltpu.VMEM((B_PER_W,), jnp.int32),
        pltpu.VMEM((B_PER_W,), jnp.int32),
        pltpu.VMEM((DIM * B_PER_W,), jnp.float32),
        pltpu.VMEM((DIM * B_PER_W,), jnp.float32),
        pltpu.VMEM((B_PER_W,), jnp.float32),
        pltpu.SemaphoreType.DMA,
    ],
)
def _sc_kernel(uin_hbm, iin_hbm, u0, u1, u2, u3, i0, i1, i2, i3, out_hbm,
               uidx, iidx, ucomp, icomp, outv, sem):
    wid = lax.axis_index("s") * 2 + lax.axis_index("c")
    base = wid * B_PER_W

    pltpu.sync_copy(uin_hbm.at[pl.ds(base, B_PER_W)], uidx)
    pltpu.sync_copy(iin_hbm.at[pl.ds(base, B_PER_W)], iidx)

    copies = []
    for d, (ut, it) in enumerate(((u0, i0), (u1, i1), (u2, i2), (u3, i3))):
        sl = pl.ds(d * B_PER_W, B_PER_W)
        copies.append(pltpu.make_async_copy(ut.at[uidx], ucomp.at[sl], sem))
        copies.append(pltpu.make_async_copy(it.at[iidx], icomp.at[sl], sem))
    for cp in copies:
        cp.start()
    for cp in copies:
        cp.wait()

    for c in range(N_VEC):
        ud = [ucomp[pl.ds(d * B_PER_W + c * LANES, LANES)] for d in range(DIM)]
        vd = [icomp[pl.ds(d * B_PER_W + c * LANES, LANES)] for d in range(DIM)]
        dot = ud[0] * vd[0]
        nu = ud[0] * ud[0]
        nv = vd[0] * vd[0]
        for d in range(1, DIM):
            dot = dot + ud[d] * vd[d]
            nu = nu + ud[d] * ud[d]
            nv = nv + vd[d] * vd[d]
        outv[pl.ds(c * LANES, LANES)] = dot * _rsqrt16(nu) * _rsqrt16(nv)

    pltpu.sync_copy(outv, out_hbm.at[pl.ds(base, B_PER_W)])


def kernel(user_input, item_input, user_table, item_table):
    ucols = [user_table[:, d] for d in range(DIM)]
    icols = [item_table[:, d] for d in range(DIM)]
    return _sc_kernel(user_input, item_input, *ucols, *icols)


# speedup vs baseline: 24.4635x; 1.4069x over previous
import functools

import jax
import jax.numpy as jnp
from jax import lax
from jax.experimental import pallas as pl
from jax.experimental.pallas import tpu as pltpu
from jax.experimental.pallas import tpu_sc as plsc

BATCH = 16384
DIM = 4
LANES = 16
NUM_WORKERS = 32
B_PER_W = BATCH // NUM_WORKERS
N_VEC = B_PER_W // LANES


def _rsqrt16(x):
    i = lax.bitcast_convert_type(x, jnp.int32)
    y = lax.bitcast_convert_type(jnp.int32(0x5F3759DF) - (i >> 1), jnp.float32)
    for _ in range(3):
        y = y * (jnp.float32(1.5) - jnp.float32(0.5) * x * y * y)
    return y


mesh = plsc.VectorSubcoreMesh(core_axis_name="c", subcore_axis_name="s")


@functools.partial(
    pl.kernel,
    mesh=mesh,
    out_type=jax.ShapeDtypeStruct((BATCH,), jnp.float32),
    compiler_params=pltpu.CompilerParams(
        needs_layout_passes=False, use_tc_tiling_on_sc=False
    ),
    scratch_types=[
        pltpu.VMEM((B_PER_W,), jnp.int32),
        pltpu.VMEM((B_PER_W,), jnp.int32),
        pltpu.VMEM((DIM * B_PER_W,), jnp.float32),
        pltpu.VMEM((DIM * B_PER_W,), jnp.float32),
        pltpu.VMEM((B_PER_W,), jnp.float32),
        pltpu.SemaphoreType.DMA,
    ],
)
def _sc_kernel(uin_hbm, iin_hbm, utab_hbm, itab_hbm, out_hbm,
               uidx, iidx, ucomp, icomp, outv, sem):
    wid = lax.axis_index("s") * 2 + lax.axis_index("c")
    base = wid * B_PER_W
    nrows = utab_hbm.shape[0] // DIM

    pltpu.sync_copy(uin_hbm.at[pl.ds(base, B_PER_W)], uidx)
    pltpu.sync_copy(iin_hbm.at[pl.ds(base, B_PER_W)], iidx)

    copies = []
    for d in range(DIM):
        sl = pl.ds(d * B_PER_W, B_PER_W)
        usrc = utab_hbm.at[pl.ds(d * nrows, nrows)]
        isrc = itab_hbm.at[pl.ds(d * nrows, nrows)]
        copies.append(pltpu.make_async_copy(usrc.at[uidx], ucomp.at[sl], sem))
        copies.append(pltpu.make_async_copy(isrc.at[iidx], icomp.at[sl], sem))
    for cp in copies:
        cp.start()
    for cp in copies:
        cp.wait()

    for c in range(N_VEC):
        ud = [ucomp[pl.ds(d * B_PER_W + c * LANES, LANES)] for d in range(DIM)]
        vd = [icomp[pl.ds(d * B_PER_W + c * LANES, LANES)] for d in range(DIM)]
        dot = ud[0] * vd[0]
        nu = ud[0] * ud[0]
        nv = vd[0] * vd[0]
        for d in range(1, DIM):
            dot = dot + ud[d] * vd[d]
            nu = nu + ud[d] * ud[d]
            nv = nv + vd[d] * vd[d]
        outv[pl.ds(c * LANES, LANES)] = dot * _rsqrt16(nu) * _rsqrt16(nv)

    pltpu.sync_copy(outv, out_hbm.at[pl.ds(base, B_PER_W)])


def kernel(user_input, item_input, user_table, item_table):
    return _sc_kernel(user_input, item_input,
                      user_table.T.reshape(-1), item_table.T.reshape(-1))
